# Initial kernel scaffold; baseline (speedup 1.0000x reference)
import jax
import jax.numpy as jnp
from jax.experimental import pallas as pl


def kernel(queries, keys, values, attn_mask):
    raise NotImplementedError("write your pallas kernel here")


# speedup vs baseline: 4.9239x; 4.9239x over previous
import math

import jax
import jax.numpy as jnp
import numpy as np
from jax import lax
from jax.experimental import pallas as pl
from jax.experimental.pallas import tpu as pltpu

_CONST_CACHE = {}


def _count_matrix(L, S, u):
    ck = (L, S, u)
    if ck not in _CONST_CACHE:
        with jax.ensure_compile_time_eval():
            idx_key = jax.random.fold_in(jax.random.key(42), 7)
            idx = np.asarray(jax.random.randint(idx_key, (L, u), 0, S))
        C = np.zeros((L, S), np.int8)
        np.add.at(C, (np.arange(L)[:, None], idx), 1)
        _CONST_CACHE[ck] = C
    return _CONST_CACHE[ck]


def _body(q_ref, k_ref, v_ref, c_ref, o_ref,
          m_ref, qsel_ref, upd_ref, mtopf_ref, midx_smem):
    L, D = q_ref.shape[1], q_ref.shape[2]
    S = k_ref.shape[1]
    U = qsel_ref.shape[0]
    LB = 256
    NB = L // LB
    neg = jnp.float32(-jnp.inf)
    scale = jnp.float32(1.0 / math.sqrt(D))
    cdims = (((1,), (1,)), ((), ()))

    kk = k_ref[0]
    vv = v_ref[0]

    for r in range(NB):
        qb = q_ref[0, LB * r:LB * (r + 1), :]
        qk = lax.dot_general(qb, kk, cdims, preferred_element_type=jnp.float32)
        cf = c_ref[LB * r:LB * (r + 1), :].astype(jnp.float32)
        mmax = jnp.max(jnp.where(cf > 0, qk, neg), axis=1)
        msum = jnp.sum(qk * cf, axis=1)
        m_ref[r, :] = mmax - msum / jnp.float32(S)

    lin = (lax.broadcasted_iota(jnp.int32, (NB, LB), 0) * LB
           + lax.broadcasted_iota(jnp.int32, (NB, LB), 1))

    def tk_body(i, mv):
        mx = jnp.max(mv)
        idxv = jnp.min(jnp.where(mv == mx, lin, S))
        midx_smem[i] = idxv
        mtopf_ref[pl.ds(i, 1), :] = jnp.broadcast_to(
            idxv.astype(jnp.float32), (1, 128))
        qsel_ref[pl.ds(i, 1), :] = q_ref[0, pl.ds(idxv, 1), :]
        return jnp.where(lin == idxv, neg, mv)

    lax.fori_loop(0, U, tk_body, m_ref[:, :])

    qsel = qsel_ref[:, :]
    sc = lax.dot_general(qsel, kk, cdims,
                         preferred_element_type=jnp.float32) * scale
    colf = lax.broadcasted_iota(jnp.int32, (U, S), 1).astype(jnp.float32)
    scm = jnp.where(colf <= mtopf_ref[:, 0:1], sc, neg)
    rmax = jnp.max(scm, axis=1, keepdims=True)
    p = jnp.exp(scm - rmax)
    attn = p / jnp.sum(p, axis=1, keepdims=True)
    upd_ref[:, :] = lax.dot_general(attn, vv, (((1,), (0,)), ((), ())),
                                    preferred_element_type=jnp.float32)

    tri = (lax.broadcasted_iota(jnp.int32, (D, D), 0)
           <= lax.broadcasted_iota(jnp.int32, (D, D), 1))
    T = jnp.where(tri, jnp.float32(1.0), jnp.float32(0.0))
    o_ref[0, :, :] = lax.dot_general(vv, T, (((1,), (0,)), ((), ())),
                                    preferred_element_type=jnp.float32)

    def sc_body(i, carry):
        t = midx_smem[i]
        o_ref[0, pl.ds(t, 1), :] = upd_ref[pl.ds(i, 1), :]
        return carry

    lax.fori_loop(0, U, sc_body, 0)


def kernel(queries, keys, values, attn_mask):
    B, L, H, D = queries.shape
    S = keys.shape[1]
    factor = 5
    U = factor * int(np.ceil(np.log(S)))
    u = factor * int(np.ceil(np.log(L)))
    C = jnp.asarray(_count_matrix(L, S, u))

    q = queries.reshape(B * H, L, D)
    k = keys.reshape(B * H, S, D)
    v = values.reshape(B * H, S, D)

    LB = 256
    NB = L // LB
    out = pl.pallas_call(
        _body,
        grid=(B * H,),
        in_specs=[
            pl.BlockSpec((1, L, D), lambda j: (j, 0, 0)),
            pl.BlockSpec((1, S, D), lambda j: (j, 0, 0)),
            pl.BlockSpec((1, S, D), lambda j: (j, 0, 0)),
            pl.BlockSpec((L, S), lambda j: (0, 0)),
        ],
        out_specs=pl.BlockSpec((1, S, D), lambda j: (j, 0, 0)),
        out_shape=jax.ShapeDtypeStruct((B * H, S, D), jnp.float32),
        scratch_shapes=[
            pltpu.VMEM((NB, LB), jnp.float32),
            pltpu.VMEM((U, D), jnp.float32),
            pltpu.VMEM((U, D), jnp.float32),
            pltpu.VMEM((U, 128), jnp.float32),
            pltpu.SMEM((U,), jnp.int32),
        ],
    )(q, k, v, C)
    return out.reshape(B, H, S, D)


# speedup vs baseline: 9.5323x; 1.9359x over previous
import math

import jax
import jax.numpy as jnp
import numpy as np
from jax import lax
from jax.experimental import pallas as pl
from jax.experimental.pallas import tpu as pltpu

_CONST_CACHE = {}


def _tf2x32(k1, k2, x0, x1):
    k1 = np.uint32(k1)
    k2 = np.uint32(k2)
    x0 = x0.astype(np.uint32).copy()
    x1 = x1.astype(np.uint32).copy()
    kx = np.uint32(k1 ^ k2 ^ np.uint32(0x1BD11BDA))
    rot1 = (13, 15, 26, 6)
    rot2 = (17, 29, 16, 24)
    ks = (k1, k2, kx)

    def rol(v, d):
        return (v << np.uint32(d)) | (v >> np.uint32(32 - d))

    with np.errstate(over="ignore"):
        x0 += ks[0]
        x1 += ks[1]
        rots = (rot1, rot2, rot1, rot2, rot1)
        for i in range(5):
            for r in rots[i]:
                x0 += x1
                x1 = rol(x1, r)
                x1 ^= x0
            x0 += ks[(i + 1) % 3]
            x1 += ks[(i + 2) % 3] + np.uint32(i + 1)
    return x0, x1


def _sample_indices(L, S, u):
    a, b = _tf2x32(np.uint32(0), np.uint32(42),
                   np.array([0], np.uint32), np.array([7], np.uint32))
    k1, k2 = a[0], b[0]
    c1, c2 = _tf2x32(k1, k2, np.zeros(2, np.uint32),
                     np.arange(2, dtype=np.uint32))
    lo_key = (c1[1], c2[1])
    n = L * u
    hi = np.zeros(n, np.uint32)
    lo = np.arange(n, dtype=np.uint32)
    b1, b2 = _tf2x32(lo_key[0], lo_key[1], hi, lo)
    bits = b1 ^ b2
    return (bits % np.uint32(S)).astype(np.int32).reshape(L, u)


def _mask_matrices(L, S, u):
    ck = (L, S, u)
    if ck not in _CONST_CACHE:
        idx = _sample_indices(L, S, u)
        C = np.zeros((L, S), np.float32)
        np.add.at(C, (np.arange(L)[:, None], idx), 1.0)
        A = np.where(C > 0, np.float32(0), np.float32(-np.inf))
        _CONST_CACHE[ck] = (C, A)
    return _CONST_CACHE[ck]


def _m_body(q_ref, k_ref, cf_ref, madd_ref, m_ref):
    L, D = q_ref.shape[1], q_ref.shape[2]
    S = k_ref.shape[1]
    LB = 256
    kk = k_ref[0]
    cdims = (((1,), (1,)), ((), ()))
    for r in range(L // LB):
        sl = slice(LB * r, LB * (r + 1))
        qk = lax.dot_general(q_ref[0, sl, :], kk, cdims,
                             preferred_element_type=jnp.float32)
        mmax = jnp.max(qk + madd_ref[sl, :], axis=1, keepdims=True)
        msum = jnp.sum(qk * cf_ref[sl, :], axis=1, keepdims=True)
        m_ref[0, sl, :] = mmax - msum / jnp.float32(S)


def _topk_body(m_ref, mtopf_ref):
    BH = m_ref.shape[0]
    S = m_ref.shape[1]
    U = mtopf_ref.shape[1]
    neg = jnp.float32(-jnp.inf)
    mv = m_ref[:, :, 0]
    lin = lax.broadcasted_iota(jnp.int32, (BH, S), 1)
    for i in range(U):
        rmax = jnp.max(mv, axis=1, keepdims=True)
        idxc = jnp.min(jnp.where(mv == rmax, lin, S), axis=1, keepdims=True)
        mtopf_ref[:, pl.ds(i, 1), :] = jnp.broadcast_to(
            idxc.astype(jnp.float32)[:, :, None], (BH, 1, 128))
        mv = jnp.where(lin == idxc, neg, mv)


def _attn_body(q_ref, k_ref, v_ref, mtopf_ref, o_ref):
    D = q_ref.shape[2]
    S = k_ref.shape[1]
    U = mtopf_ref.shape[1]
    neg = jnp.float32(-jnp.inf)
    scale = jnp.float32(1.0 / math.sqrt(D))
    kk = k_ref[0]
    vv = v_ref[0]

    mtopf = mtopf_ref[0, :, 0:1]
    colf = lax.broadcasted_iota(jnp.int32, (U, S), 1).astype(jnp.float32)
    oh = (colf == mtopf).astype(jnp.float32)

    qsel = lax.dot_general(oh, q_ref[0], (((1,), (0,)), ((), ())),
                           preferred_element_type=jnp.float32)
    sc = lax.dot_general(qsel, kk, (((1,), (1,)), ((), ())),
                         preferred_element_type=jnp.float32) * scale
    scm = jnp.where(colf <= mtopf, sc, neg)
    rmax = jnp.max(scm, axis=1, keepdims=True)
    p = jnp.exp(scm - rmax)
    attn = p / jnp.sum(p, axis=1, keepdims=True)
    upd = lax.dot_general(attn, vv, (((1,), (0,)), ((), ())),
                          preferred_element_type=jnp.float32)

    tri = (lax.broadcasted_iota(jnp.int32, (D, D), 0)
           <= lax.broadcasted_iota(jnp.int32, (D, D), 1))
    T = jnp.where(tri, jnp.float32(1.0), jnp.float32(0.0))
    cumv = lax.dot_general(vv, T, (((1,), (0,)), ((), ())),
                           preferred_element_type=jnp.float32)

    merged = jnp.concatenate([upd, jnp.ones((U, 1), jnp.float32)], axis=1)
    full = lax.dot_general(oh, merged, (((0,), (0,)), ((), ())),
                           preferred_element_type=jnp.float32)
    o_ref[0, 0, :, :] = jnp.where(full[:, D:D + 1] > 0.5, full[:, :D], cumv)


def kernel(queries, keys, values, attn_mask):
    B, L, H, D = queries.shape
    S = keys.shape[1]
    factor = 5
    U = factor * int(np.ceil(np.log(S)))
    u = factor * int(np.ceil(np.log(L)))
    Cnp, Anp = _mask_matrices(L, S, u)
    CF = jnp.asarray(Cnp)
    MADD = jnp.asarray(Anp)

    q = queries.reshape(B * H, L, D)
    k = keys.reshape(B * H, S, D)
    v = values.reshape(B * H, S, D)

    m = pl.pallas_call(
        _m_body,
        grid=(B * H,),
        in_specs=[
            pl.BlockSpec((1, L, D), lambda j: (j, 0, 0)),
            pl.BlockSpec((1, S, D), lambda j: (j, 0, 0)),
            pl.BlockSpec((L, S), lambda j: (0, 0)),
            pl.BlockSpec((L, S), lambda j: (0, 0)),
        ],
        out_specs=pl.BlockSpec((1, L, 1), lambda j: (j, 0, 0)),
        out_shape=jax.ShapeDtypeStruct((B * H, L, 1), jnp.float32),
    )(q, k, CF, MADD)

    mtopf = pl.pallas_call(
        _topk_body,
        in_specs=[pl.BlockSpec((B * H, L, 1), lambda: (0, 0, 0))],
        out_specs=pl.BlockSpec((B * H, U, 128), lambda: (0, 0, 0)),
        out_shape=jax.ShapeDtypeStruct((B * H, U, 128), jnp.float32),
    )(m)

    out = pl.pallas_call(
        _attn_body,
        grid=(B * H,),
        in_specs=[
            pl.BlockSpec((1, L, D), lambda j: (j, 0, 0)),
            pl.BlockSpec((1, S, D), lambda j: (j, 0, 0)),
            pl.BlockSpec((1, S, D), lambda j: (j, 0, 0)),
            pl.BlockSpec((1, U, 128), lambda j: (j, 0, 0)),
        ],
        out_specs=pl.BlockSpec((1, 1, S, D),
                               lambda j: (j // H, j % H, 0, 0)),
        out_shape=jax.ShapeDtypeStruct((B, H, S, D), jnp.float32),
    )(q, k, v, mtopf)
    return out


# speedup vs baseline: 9.8183x; 1.0300x over previous
import math

import jax
import jax.numpy as jnp
import ml_dtypes
import numpy as np
from jax import lax
from jax.experimental import pallas as pl
from jax.experimental.pallas import tpu as pltpu

_CONST_CACHE = {}


def _tf2x32(k1, k2, x0, x1):
    k1 = np.uint32(k1)
    k2 = np.uint32(k2)
    x0 = x0.astype(np.uint32).copy()
    x1 = x1.astype(np.uint32).copy()
    kx = np.uint32(k1 ^ k2 ^ np.uint32(0x1BD11BDA))
    rot1 = (13, 15, 26, 6)
    rot2 = (17, 29, 16, 24)
    ks = (k1, k2, kx)

    def rol(v, d):
        return (v << np.uint32(d)) | (v >> np.uint32(32 - d))

    with np.errstate(over="ignore"):
        x0 += ks[0]
        x1 += ks[1]
        rots = (rot1, rot2, rot1, rot2, rot1)
        for i in range(5):
            for r in rots[i]:
                x0 += x1
                x1 = rol(x1, r)
                x1 ^= x0
            x0 += ks[(i + 1) % 3]
            x1 += ks[(i + 2) % 3] + np.uint32(i + 1)
    return x0, x1


def _sample_indices(L, S, u):
    a, b = _tf2x32(np.uint32(0), np.uint32(42),
                   np.array([0], np.uint32), np.array([7], np.uint32))
    k1, k2 = a[0], b[0]
    c1, c2 = _tf2x32(k1, k2, np.zeros(2, np.uint32),
                     np.arange(2, dtype=np.uint32))
    lo_key = (c1[1], c2[1])
    n = L * u
    hi = np.zeros(n, np.uint32)
    lo = np.arange(n, dtype=np.uint32)
    b1, b2 = _tf2x32(lo_key[0], lo_key[1], hi, lo)
    bits = b1 ^ b2
    return (bits % np.uint32(S)).astype(np.int32).reshape(L, u)


def _mask_matrices(L, S, u):
    ck = (L, S, u)
    if ck not in _CONST_CACHE:
        idx = _sample_indices(L, S, u)
        C = np.zeros((L, S), np.float32)
        np.add.at(C, (np.arange(L)[:, None], idx), 1.0)
        A = np.where(C > 0, np.float32(0), np.float32(-np.inf))
        _CONST_CACHE[ck] = (C.T.copy().astype(ml_dtypes.bfloat16),
                            A.T.copy().astype(ml_dtypes.bfloat16))
    return _CONST_CACHE[ck]


def _make_body(BH, U):
    def _body(q_ref, k_ref, v_ref, cf_ref, madd_ref, o_ref, m_sc, mt_sc):
        L, D = q_ref.shape[1], q_ref.shape[2]
        S = k_ref.shape[1]
        LB = 256
        neg = jnp.float32(-jnp.inf)
        scale = jnp.float32(1.0 / math.sqrt(D))
        pid = pl.program_id(0)

        @pl.when(pid < BH)
        def _phase_m():
            kk = k_ref[0]
            cdims = (((1,), (1,)), ((), ()))
            for r in range(L // LB):
                sl = slice(LB * r, LB * (r + 1))
                qkT = lax.dot_general(kk, q_ref[0, sl, :], cdims,
                                      preferred_element_type=jnp.float32)
                madd = madd_ref[:, sl].astype(jnp.float32)
                cf = cf_ref[:, sl].astype(jnp.float32)
                mmax = jnp.max(qkT + madd, axis=0, keepdims=True)
                msum = jnp.sum(qkT * cf, axis=0, keepdims=True)
                m_sc[pl.ds(pid, 1), sl] = mmax - msum / jnp.float32(S)

        @pl.when(pid == BH)
        def _phase_topk():
            mv = m_sc[:, :]
            lin = lax.broadcasted_iota(jnp.int32, (BH, S), 1)
            for i in range(U):
                rmax = jnp.max(mv, axis=1, keepdims=True)
                idxc = jnp.min(jnp.where(mv == rmax, lin, S), axis=1,
                               keepdims=True)
                mt_sc[:, pl.ds(i, 1), :] = jnp.broadcast_to(
                    idxc.astype(jnp.float32)[:, :, None], (BH, 1, 128))
                mv = jnp.where(lin == idxc, neg, mv)

        @pl.when(pid > BH)
        def _phase_attn():
            bh = pid - BH - 1
            kk = k_ref[0]
            vv = v_ref[0]
            mtopf = mt_sc[pl.ds(bh, 1), :, 0:1].reshape(U, 1)
            colf = lax.broadcasted_iota(
                jnp.int32, (U, S), 1).astype(jnp.float32)
            oh = (colf == mtopf).astype(jnp.float32)

            qsel = lax.dot_general(oh, q_ref[0], (((1,), (0,)), ((), ())),
                                   preferred_element_type=jnp.float32)
            sc = lax.dot_general(qsel, kk, (((1,), (1,)), ((), ())),
                                 preferred_element_type=jnp.float32) * scale
            scm = jnp.where(colf <= mtopf, sc, neg)
            rmax = jnp.max(scm, axis=1, keepdims=True)
            p = jnp.exp(scm - rmax)
            attn = p / jnp.sum(p, axis=1, keepdims=True)
            upd = lax.dot_general(attn, vv, (((1,), (0,)), ((), ())),
                                  preferred_element_type=jnp.float32)

            tri = (lax.broadcasted_iota(jnp.int32, (D, D), 0)
                   <= lax.broadcasted_iota(jnp.int32, (D, D), 1))
            T = jnp.where(tri, jnp.float32(1.0), jnp.float32(0.0))
            cumv = lax.dot_general(vv, T, (((1,), (0,)), ((), ())),
                                   preferred_element_type=jnp.float32)

            merged = jnp.concatenate(
                [upd, jnp.ones((U, 1), jnp.float32)], axis=1)
            full = lax.dot_general(oh, merged, (((0,), (0,)), ((), ())),
                                   preferred_element_type=jnp.float32)
            o_ref[0, 0, :, :] = jnp.where(
                full[:, D:D + 1] > 0.5, full[:, :D], cumv)

    return _body


def kernel(queries, keys, values, attn_mask):
    B, L, H, D = queries.shape
    S = keys.shape[1]
    BH = B * H
    factor = 5
    U = factor * int(np.ceil(np.log(S)))
    u = factor * int(np.ceil(np.log(L)))
    Cnp, Anp = _mask_matrices(L, S, u)
    CF = jnp.asarray(Cnp)
    MADD = jnp.asarray(Anp)

    q = queries.reshape(BH, L, D)
    k = keys.reshape(BH, S, D)
    v = values.reshape(BH, S, D)

    def bh_map(j):
        return jnp.where(j < BH, j, jnp.maximum(j - BH - 1, 0))

    out = pl.pallas_call(
        _make_body(BH, U),
        grid=(2 * BH + 1,),
        in_specs=[
            pl.BlockSpec((1, L, D), lambda j: (bh_map(j), 0, 0)),
            pl.BlockSpec((1, S, D), lambda j: (bh_map(j), 0, 0)),
            pl.BlockSpec((1, S, D), lambda j: (bh_map(j), 0, 0)),
            pl.BlockSpec((S, L), lambda j: (0, 0)),
            pl.BlockSpec((S, L), lambda j: (0, 0)),
        ],
        out_specs=pl.BlockSpec(
            (1, 1, S, D),
            lambda j: (bh_map(j) // H, bh_map(j) % H, 0, 0)),
        out_shape=jax.ShapeDtypeStruct((B, H, S, D), jnp.float32),
        scratch_shapes=[
            pltpu.VMEM((BH, L), jnp.float32),
            pltpu.VMEM((BH, U, 128), jnp.float32),
        ],
    )(q, k, v, CF, MADD)
    return out
